# baseline (speedup 1.0000x reference)
import functools

import jax
import jax.numpy as jnp
from jax import lax
from jax.experimental import pallas as pl
from jax.experimental.pallas import tpu as pltpu
from jax.experimental.pallas import tpu_sc as plsc

BATCH = 16384
EDIM = 64
ROW = 2 * EDIM
NC = 2
NS = 16
NW = NC * NS
CHUNK = 128
NSPLIT = 2

_f32 = jnp.float32


def _pack_body(umf_t, umlp_t, imf_t, imlp_t, out_u, out_i):
  out_u[:, :EDIM] = umf_t[...].T
  out_u[:, EDIM:] = umlp_t[...].T
  out_i[:, :EDIM] = imf_t[...].T
  out_i[:, EDIM:] = imlp_t[...].T


def _pack_tables(user_mf, user_mlp, item_mf, item_mlp):
  v = user_mf.shape[0]
  tab = jax.ShapeDtypeStruct((v, ROW), _f32)
  return pl.pallas_call(
      _pack_body,
      out_shape=[tab, tab],
  )(user_mf.T, user_mlp.T, item_mf.T, item_mlp.T)


def _sc_gather_body(nch, user_hbm, item_hbm, utab, itab, out_u, out_i,
                    idx_u, idx_i, buf, sem):
  wid = lax.axis_index("s") * NC + lax.axis_index("c")
  r0 = wid * nch
  pltpu.sync_copy(user_hbm.at[pl.ds(r0, nch)], idx_u)
  pltpu.sync_copy(item_hbm.at[pl.ds(r0, nch)], idx_i)

  def gather(tab, idx, dst):
    copies = []
    for j in range(nch):
      copies.append(pltpu.async_copy(tab.at[idx.at[j]], buf.at[j], sem))
    for c in copies:
      c.wait()
    pltpu.sync_copy(buf, dst.at[pl.ds(r0, nch)])

  gather(utab, idx_u, out_u)
  gather(itab, idx_i, out_i)


def _sc_gather(user2d, item2d, utab, itab):
  nchunks = user2d.shape[0]
  nch = nchunks // NW
  mesh = plsc.VectorSubcoreMesh(core_axis_name="c", subcore_axis_name="s")
  out3 = jax.ShapeDtypeStruct((nchunks, CHUNK, ROW), _f32)
  fn = functools.partial(
      pl.kernel,
      mesh=mesh,
      out_type=[out3, out3],
      scratch_types=[
          pltpu.VMEM((nch, CHUNK), jnp.int32),
          pltpu.VMEM((nch, CHUNK), jnp.int32),
          pltpu.VMEM((nch, CHUNK, ROW), _f32),
          pltpu.SemaphoreType.DMA,
      ],
      compiler_params=pltpu.CompilerParams(use_tc_tiling_on_sc=False),
  )(functools.partial(_sc_gather_body, nch))
  return fn(user2d, item2d, utab, itab)


def _tc_body(uref, iref, w1, b1, w2, b2, wpm, wph, bp, out):
  u = uref[...]
  i = iref[...]
  mf = u[:, :EDIM] * i[:, :EDIM]
  h1 = (lax.dot_general(u[:, EDIM:], w1[:, :EDIM], (((1,), (1,)), ((), ())),
                        preferred_element_type=_f32)
        + lax.dot_general(i[:, EDIM:], w1[:, EDIM:], (((1,), (1,)), ((), ())),
                          preferred_element_type=_f32))
  h1 = jnp.maximum(h1 + b1[...], 0.0)
  h2 = lax.dot_general(h1, w2[...], (((1,), (1,)), ((), ())),
                       preferred_element_type=_f32)
  h2 = jnp.maximum(h2 + b2[...], 0.0)
  logit = (jnp.sum(mf * wpm[...], axis=1, keepdims=True)
           + jnp.sum(h2 * wph[...], axis=1, keepdims=True)
           + bp[...])
  out[...] = jax.nn.sigmoid(logit).reshape(out.shape)


def _tc_mlp(urows, irows, W1, b1, W2, b2, wpm, wph, bp):
  batch = urows.shape[0]
  blk = 2048
  grid = batch // blk
  row_spec = pl.BlockSpec((blk, ROW), lambda i: (i, 0))
  full = lambda shape: pl.BlockSpec(shape, lambda i: (0, 0))
  out2 = pl.pallas_call(
      _tc_body,
      grid=(grid,),
      in_specs=[row_spec, row_spec,
                full((128, 128)), full((1, 128)),
                full((64, 128)), full((1, 64)),
                full((1, 64)), full((1, 64)), full((1, 1))],
      out_specs=pl.BlockSpec((1, 1, blk), lambda i: (i, 0, 0)),
      out_shape=jax.ShapeDtypeStruct((grid, 1, blk), _f32),
  )(urows, irows, W1, b1, W2, b2, wpm, wph, bp)
  return out2.reshape(batch)


def kernel(user, item, user_mf, item_mf, user_mlp, item_mlp,
           W1, b1, W2, b2, Wp, bp):
  user2d = user.astype(jnp.int32).reshape(BATCH // CHUNK, CHUNK)
  item2d = item.astype(jnp.int32).reshape(BATCH // CHUNK, CHUNK)
  utab, itab = _pack_tables(user_mf, user_mlp, item_mf, item_mlp)
  wp = Wp.reshape(128)
  wpm = wp[:EDIM].reshape(1, EDIM)
  wph = wp[EDIM:].reshape(1, EDIM)
  b1r = b1.reshape(1, 128)
  b2r = b2.reshape(1, 64)
  bpr = bp.reshape(1, 1)

  nchunks = BATCH // CHUNK
  step = nchunks // NSPLIT
  outs = []
  for s in range(NSPLIT):
    u2 = lax.slice_in_dim(user2d, s * step, (s + 1) * step, axis=0)
    i2 = lax.slice_in_dim(item2d, s * step, (s + 1) * step, axis=0)
    urows3, irows3 = _sc_gather(u2, i2, utab, itab)
    urows = urows3.reshape(step * CHUNK, ROW)
    irows = irows3.reshape(step * CHUNK, ROW)
    outs.append(_tc_mlp(urows, irows, W1, b1r, W2, b2r, wpm, wph, bpr))
  return jnp.concatenate(outs)


# speedup vs baseline: 1.1169x; 1.1169x over previous
import functools

import jax
import jax.numpy as jnp
from jax import lax
from jax.experimental import pallas as pl
from jax.experimental.pallas import tpu as pltpu
from jax.experimental.pallas import tpu_sc as plsc

BATCH = 16384
EDIM = 64
ROW = 2 * EDIM
NC = 2
NS = 16
NW = NC * NS
CHUNK = 128
NCHUNKS = BATCH // CHUNK
NCH = NCHUNKS // NW
LANES = 16
NG = EDIM // LANES

_f32 = jnp.float32


def _pack_body(umf_t, umlp_t, imf_t, imlp_t, wpm, out_u, out_i):
  out_u[:, :EDIM] = umf_t[...].T * wpm[...]
  out_u[:, EDIM:] = umlp_t[...].T
  out_i[:, :EDIM] = imf_t[...].T
  out_i[:, EDIM:] = imlp_t[...].T


def _pack_tables(user_mf, user_mlp, item_mf, item_mlp, wpm):
  v = user_mf.shape[0]
  tab = jax.ShapeDtypeStruct((v, ROW), _f32)
  return pl.pallas_call(
      _pack_body,
      out_shape=[tab, tab],
  )(user_mf.T, user_mlp.T, item_mf.T, item_mlp.T, wpm)


def _sc_gather_body(user_hbm, item_hbm, utab, itab, out_x, out_mfp,
                    idx_u, idx_i, buf_u, buf_i, mfp_v, sem, wsem):
  wid = lax.axis_index("s") * NC + lax.axis_index("c")
  r0 = wid * NCH
  pltpu.sync_copy(user_hbm.at[pl.ds(r0, NCH)], idx_u)
  pltpu.sync_copy(item_hbm.at[pl.ds(r0, NCH)], idx_i)
  i16 = lax.iota(jnp.int32, LANES)
  lane_hi = i16 // 8
  lane_lo = i16 % 8
  zero16 = i16 * 0

  def fire_gather(j):
    s = j % 2
    return (pltpu.async_copy(utab.at[idx_u.at[j]], buf_u.at[s], sem),
            pltpu.async_copy(itab.at[idx_i.at[j]], buf_i.at[s], sem))

  def fire_writes(j):
    s = j % 2
    return (pltpu.async_copy(buf_u.at[s, :, pl.ds(EDIM, EDIM)],
                             out_x.at[r0 + j, :, pl.ds(0, EDIM)], wsem),
            pltpu.async_copy(buf_i.at[s, :, pl.ds(EDIM, EDIM)],
                             out_x.at[r0 + j, :, pl.ds(EDIM, EDIM)], wsem))

  g = {0: fire_gather(0)}
  if NCH > 1:
    g[1] = fire_gather(1)
  w = {}
  for j in range(NCH):
    s = j % 2
    g[j][0].wait()
    g[j][1].wait()
    if j >= 1 and j + 1 < NCH:
      w[j - 1][0].wait()
      w[j - 1][1].wait()
      g[j + 1] = fire_gather(j + 1)

    def mf_row(r, c):
      acc = (buf_u[s, r, pl.ds(0, LANES)] * buf_i[s, r, pl.ds(0, LANES)])
      for k in range(1, NG):
        acc += (buf_u[s, r, pl.ds(k * LANES, LANES)]
                * buf_i[s, r, pl.ds(k * LANES, LANES)])
      plsc.store_scatter(mfp_v, [i16 + j * LANES, zero16 + r], acc)
      return c

    lax.fori_loop(0, CHUNK, mf_row, 0, unroll=4)
    w[j] = fire_writes(j)
  tail = pltpu.async_copy(mfp_v, out_mfp.at[pl.ds(r0 * LANES, NCH * LANES)], wsem)
  for j in range(max(0, NCH - 2), NCH):
    w[j][0].wait()
    w[j][1].wait()
  tail.wait()


def _sc_gather(user2d, item2d, utab, itab):
  mesh = plsc.VectorSubcoreMesh(core_axis_name="c", subcore_axis_name="s")
  fn = functools.partial(
      pl.kernel,
      mesh=mesh,
      out_type=[jax.ShapeDtypeStruct((NCHUNKS, CHUNK, ROW), _f32),
                jax.ShapeDtypeStruct((NCHUNKS * LANES, CHUNK), _f32)],
      scratch_types=[
          pltpu.VMEM((NCH, CHUNK), jnp.int32),
          pltpu.VMEM((NCH, CHUNK), jnp.int32),
          pltpu.VMEM((2, CHUNK, ROW), _f32),
          pltpu.VMEM((2, CHUNK, ROW), _f32),
          pltpu.VMEM((NCH * LANES, CHUNK), _f32),
          pltpu.SemaphoreType.DMA,
          pltpu.SemaphoreType.DMA,
      ],
      compiler_params=pltpu.CompilerParams(use_tc_tiling_on_sc=False,
                                           needs_layout_passes=False),
  )(_sc_gather_body)
  return fn(user2d, item2d, utab, itab)


def _tc_body(xref, mfpref, w1, b1, w2, b2, wph, bp, out):
  x = xref[...]
  h1 = lax.dot_general(x, w1[...], (((1,), (1,)), ((), ())),
                       preferred_element_type=_f32)
  h1 = jnp.maximum(h1 + b1[...], 0.0)
  h2 = lax.dot_general(h1, w2[...], (((1,), (1,)), ((), ())),
                       preferred_element_type=_f32)
  h2 = jnp.maximum(h2 + b2[...], 0.0)
  s = jnp.sum(h2 * wph[...], axis=1, keepdims=True)
  m4 = mfpref[...]
  smf = jnp.sum(jnp.sum(m4, axis=1), axis=1)
  nrow = out.shape[1]
  logit = s.reshape(nrow, CHUNK) + smf + bp[...]
  out[...] = jax.nn.sigmoid(logit).reshape(out.shape)


def _tc_mlp(xrows, mfp, W1, b1, W2, b2, wph, bp):
  blk = 2048
  grid = BATCH // blk
  out2 = pl.pallas_call(
      _tc_body,
      grid=(grid,),
      in_specs=[pl.BlockSpec((blk, ROW), lambda i: (i, 0)),
                pl.BlockSpec((blk // CHUNK, 2, 8, CHUNK),
                             lambda i: (i, 0, 0, 0)),
                pl.BlockSpec((128, 128), lambda i: (0, 0)),
                pl.BlockSpec((1, 128), lambda i: (0, 0)),
                pl.BlockSpec((64, 128), lambda i: (0, 0)),
                pl.BlockSpec((1, 64), lambda i: (0, 0)),
                pl.BlockSpec((1, 64), lambda i: (0, 0)),
                pl.BlockSpec((1, 1), lambda i: (0, 0))],
      out_specs=pl.BlockSpec((1, blk // CHUNK, CHUNK), lambda i: (i, 0, 0)),
      out_shape=jax.ShapeDtypeStruct((grid, blk // CHUNK, CHUNK), _f32),
  )(xrows, mfp, W1, b1, W2, b2, wph, bp)
  return out2.reshape(BATCH)


def kernel(user, item, user_mf, item_mf, user_mlp, item_mlp,
           W1, b1, W2, b2, Wp, bp):
  user2d = user.astype(jnp.int32).reshape(NCHUNKS, CHUNK)
  item2d = item.astype(jnp.int32).reshape(NCHUNKS, CHUNK)
  wp = Wp.reshape(128)
  wpm = wp[:EDIM].reshape(1, EDIM)
  wph = wp[EDIM:].reshape(1, EDIM)
  utab, itab = _pack_tables(user_mf, user_mlp, item_mf, item_mlp, wpm)
  xrows3, mfp2 = _sc_gather(user2d, item2d, utab, itab)
  mfp4 = mfp2.reshape(NCHUNKS, 2, 8, CHUNK)
  xrows = xrows3.reshape(BATCH, ROW)
  return _tc_mlp(xrows, mfp4, W1, b1.reshape(1, 128), W2, b2.reshape(1, 64),
                 wph, bp.reshape(1, 1))
